# baseline (speedup 1.0000x reference)
import functools

import jax
import jax.numpy as jnp
import numpy as np
from jax import lax
from jax.experimental import pallas as pl
from jax.experimental.pallas import tpu as pltpu
from jax.experimental.pallas import tpu_sc as plsc

_N = 10000
_E = 320000
_D = 128
_HEADS = 8
_HID = 16

_NC = 2
_NS = 16
_CH = 80
_NCHUNK = _E // _CH
_CHT = _NCHUNK // _NS
_NPAIR = _CHT // 2
_RB = 80
_NRB = _N // _RB
_ND1 = _N // 4
_RBD1 = _RB // 4
_ND2 = _N // 8
_RBD2 = _RB // 8


def _mm_body(x_ref, w_ref, b_ref, o_ref):
  o_ref[...] = (
      jnp.dot(x_ref[...], w_ref[...], preferred_element_type=jnp.float32)
      + b_ref[...]
  )


def _matmul(x, w, b, block_rows):
  m, kdim = x.shape
  n = w.shape[1]
  return pl.pallas_call(
      _mm_body,
      grid=(m // block_rows,),
      in_specs=[
          pl.BlockSpec((block_rows, kdim), lambda i: (i, 0)),
          pl.BlockSpec((kdim, n), lambda i: (0, 0)),
          pl.BlockSpec((1, n), lambda i: (0, 0)),
      ],
      out_specs=pl.BlockSpec((block_rows, n), lambda i: (i, 0)),
      out_shape=jax.ShapeDtypeStruct((m, n), jnp.float32),
  )(x, w, b.reshape(1, n))


def _mm3_body(x_ref, w_ref, b_ref, o_ref):
  o_ref[0] = (
      jnp.dot(x_ref[...], w_ref[0], preferred_element_type=jnp.float32)
      + b_ref[0]
  )


def _ee_stacked_matmul(e, wst, bst, block_rows):
  m = e.shape[0]
  w_ = wst.shape[2]

  return pl.pallas_call(
      _mm3_body,
      grid=(_NC, m // block_rows),
      in_specs=[
          pl.BlockSpec((block_rows, 16), lambda g, i: (i, 0)),
          pl.BlockSpec((1, 16, w_), lambda g, i: (g, 0, 0)),
          pl.BlockSpec((1, 1, w_), lambda g, i: (g, 0, 0)),
      ],
      out_specs=pl.BlockSpec((1, block_rows, w_), lambda g, i: (g, i, 0)),
      out_shape=jax.ShapeDtypeStruct((_NC, m, w_), jnp.float32),
  )(e, wst, bst)


def _ep_body(u_ref, d_ref, s_ref, bm_ref, w_ref, b_ref, o_ref):
  dinv = 1.0 / jnp.maximum(d_ref[...], 1e-30)
  dbc = jnp.dot(dinv, bm_ref[...], preferred_element_type=jnp.float32)
  h = jnp.maximum(u_ref[...] * dbc + s_ref[...], 0.0)
  o_ref[...] = (
      jnp.dot(h, w_ref[...], preferred_element_type=jnp.float32) + b_ref[...]
  )


def _epilogue_proj(u, d, s, bmat, w, b, block_rows):
  m = u.shape[0]
  n = w.shape[1]
  return pl.pallas_call(
      _ep_body,
      grid=(m // block_rows,),
      in_specs=[
          pl.BlockSpec((block_rows, _D), lambda i: (i, 0)),
          pl.BlockSpec((block_rows, 16), lambda i: (i, 0)),
          pl.BlockSpec((block_rows, _D), lambda i: (i, 0)),
          pl.BlockSpec((16, _D), lambda i: (0, 0)),
          pl.BlockSpec((_D, n), lambda i: (0, 0)),
          pl.BlockSpec((1, n), lambda i: (0, 0)),
      ],
      out_specs=pl.BlockSpec((block_rows, n), lambda i: (i, 0)),
      out_shape=jax.ShapeDtypeStruct((m, n), jnp.float32),
  )(u, d, s, bmat, w, b.reshape(1, n))


def _ep_final_body(u_ref, d_ref, s_ref, bm_ref, o_ref):
  dinv = 1.0 / jnp.maximum(d_ref[...], 1e-30)
  dbc = jnp.dot(dinv, bm_ref[...], preferred_element_type=jnp.float32)
  o_ref[...] = jnp.maximum(u_ref[...] * dbc + s_ref[...], 0.0)


def _epilogue_final(u, d, s, bmat, block_rows):
  m = u.shape[0]
  return pl.pallas_call(
      _ep_final_body,
      grid=(m // block_rows,),
      in_specs=[
          pl.BlockSpec((block_rows, _D), lambda i: (i, 0)),
          pl.BlockSpec((block_rows, 16), lambda i: (i, 0)),
          pl.BlockSpec((block_rows, _D), lambda i: (i, 0)),
          pl.BlockSpec((16, _D), lambda i: (0, 0)),
      ],
      out_specs=pl.BlockSpec((block_rows, _D), lambda i: (i, 0)),
      out_shape=jax.ShapeDtypeStruct((m, _D), jnp.float32),
  )(u, d, s, bmat)


_SC_PARAMS = pltpu.CompilerParams(
    needs_layout_passes=False, use_tc_tiling_on_sc=False)
_SC_MESH = plsc.VectorSubcoreMesh(
    core_axis_name="c", subcore_axis_name="s",
    num_cores=_NC, num_subcores=_NS)
def _zero_and_plan(sid, ubuf, dbuf, acc_u, acc_d, rbd):
  zv = jnp.zeros((16,), jnp.float32)

  def zero_row(i, carry):
    for blk in range(64 // 16):
      ubuf[i, pl.ds(16 * blk, 16)] = zv
    dbuf[i, :] = zv
    return carry

  lax.fori_loop(0, _CH, zero_row, 0)
  nblk = jnp.where(sid < _NRB % _NS, _NRB // _NS + 1, _NRB // _NS)

  def zero_blk(j, carry):
    row = pl.multiple_of((sid + j * _NS) * _RB, _RB)
    rowd = pl.multiple_of((sid + j * _NS) * rbd, rbd)
    pltpu.sync_copy(ubuf.at[pl.ds(0, _RB)], acc_u.at[pl.ds(row, _RB)])
    pltpu.sync_copy(dbuf.at[pl.ds(0, rbd)], acc_d.at[pl.ds(rowd, rbd)])
    return carry

  lax.fori_loop(0, nblk, zero_blk, 0)
  plsc.subcore_barrier()
  return nblk


def _copy_out(cid, sid, nblk, acc_u, acc_d, out_u, out_d, rbd):
  plsc.subcore_barrier()

  def out_blk(j, carry):
    row = pl.multiple_of((sid + j * _NS) * _RB, _RB)
    rowd = pl.multiple_of((sid + j * _NS) * rbd, rbd)
    sl = pl.ds(row, _RB)
    sld = pl.ds(rowd, rbd)
    pltpu.sync_copy(acc_u.at[sl], out_u.at[cid, sl])
    pltpu.sync_copy(acc_d.at[sld], out_d.at[cid, sld])
    return carry

  lax.fori_loop(0, nblk, out_blk, 0)


def _halve_ids2(blk, j, db2, shift):
  for t in range(_CH // 16):
    sl = pl.ds(16 * t, 16)
    db2[0, sl] = lax.shift_right_logical(blk[j, sl], shift)


def _dst_slot_vec2(blk, j, i, m):
  dv = plsc.load_gather(
      blk, [jnp.full((16,), j, jnp.int32), jnp.full((16,), i, jnp.int32)])
  return lax.bitwise_and(dv, m)


def _xor_perms(lane):
  return tuple(lax.bitwise_xor(lane, c) for c in (8, 4, 2, 1))


def _splat_sum(t, perms):
  for p in perms:
    t = t + t.at[p].get(mode="promise_in_bounds")
  return t


@functools.partial(
    pl.kernel,
    name="sc_edge_l1",
    compiler_params=_SC_PARAMS,
    out_type=[
        jax.ShapeDtypeStruct((_NC, _N, 64), jnp.float32),
        jax.ShapeDtypeStruct((_NC, _ND1, 16), jnp.float32),
    ],
    mesh=_SC_MESH,
    scratch_types=[
        pltpu.VMEM((_CH, 64), jnp.float32),
        pltpu.VMEM((_CH, _D), jnp.float32),
        pltpu.VMEM((_CH, 64), jnp.float32),
        pltpu.VMEM((_CH, 64), jnp.float32),
        pltpu.VMEM((_CH, _D), jnp.float32),
        pltpu.VMEM((_CH, 64), jnp.float32),
        pltpu.VMEM((1, _CH), jnp.int32),
        pltpu.VMEM((1, _CH), jnp.int32),
        pltpu.VMEM((1, _CH), jnp.int32),
        pltpu.VMEM((1, _CH), jnp.int32),
        pltpu.VMEM((1, _CH), jnp.int32),
        pltpu.VMEM((_CH, 64), jnp.float32),
        pltpu.VMEM((_CH, 16), jnp.float32),
        pltpu.VMEM_SHARED((_N, 64), jnp.float32),
        pltpu.VMEM_SHARED((_ND1, 16), jnp.float32),
    ] + [pltpu.SemaphoreType.DMA] * 6,
)
def _sc_edge_l1(q_hbm, kv_hbm, ee_hbm, src_hbm, dst_hbm,
                out_u, out_d,
                qb0, kvb0, eb0, qb1, kvb1, eb1,
                sb0, db0, sb1, db1, dh, ubuf, dbuf, acc_u, acc_d,
                mq0, mk0, me0, mq1, mk1, me1):
  cid = lax.axis_index("c")
  sid = lax.axis_index("s")
  nblk = _zero_and_plan(sid, ubuf, dbuf, acc_u, acc_d, _RBD1)
  lane = lax.iota(jnp.int32, 16)
  lanem = lax.bitwise_and(lane, 3)
  laneq = lax.shift_right_logical(lane, 2)
  perms = _xor_perms(lane)
  zv = jnp.zeros((16,), jnp.float32)

  first = pl.multiple_of(sid * _CHT, 2)

  bufs = ((qb0, kvb0, eb0, sb0, db0, mq0, mk0, me0),
          (qb1, kvb1, eb1, sb1, db1, mq1, mk1, me1))

  def prefetch(j, b):
    qb, kvb, eb, sb, db_, mq, mk, me = bufs[b]
    jc = jnp.minimum(j, _CHT - 1)
    base = pl.multiple_of((first + jc) * _CH, 16)
    pltpu.sync_copy(src_hbm.at[pl.ds(first + jc, 1)], sb)
    pltpu.sync_copy(dst_hbm.at[pl.ds(first + jc, 1)], db_)
    pltpu.async_copy(q_hbm.at[cid].at[db_.at[0]], qb, mq)
    pltpu.async_copy(kv_hbm.at[cid].at[sb.at[0]], kvb, mk)
    pltpu.async_copy(ee_hbm.at[cid].at[pl.ds(base, _CH)], eb, me)

  def drain(b):
    qb, kvb, eb, sb, db_, mq, mk, me = bufs[b]
    dummy_q = q_hbm.at[cid].at[pl.ds(0, _CH)]
    dummy_kv = kv_hbm.at[cid].at[pl.ds(0, _CH)]
    pltpu.make_async_copy(dummy_q, qb, mq).wait()
    pltpu.make_async_copy(dummy_kv, kvb, mk).wait()
    pltpu.make_async_copy(dummy_q, eb, me).wait()

  def consume(j, b):
    qb, kvb, eb, sb, db_, mq, mk, me = bufs[b]
    drain(b)

    def edge_body(i, ecarry):
      dacc = zv
      for h in range(_HEADS // _NC):
        sl = pl.ds(16 * h, 16)
        slv = pl.ds(64 + 16 * h, 16)
        kj = kvb[i, sl] + eb[i, sl]
        aev = jnp.exp(_splat_sum(qb[i, sl] * kj, perms))
        ubuf[i, sl] = (kvb[i, slv] + eb[i, sl]) * aev
        dacc = dacc + jnp.where(lanem == h, aev, 0.0)
      pv = _dst_slot_vec2(db_, 0, i, 3)
      dbuf[i, :] = jnp.where(laneq == pv, dacc, 0.0)
      return ecarry

    lax.fori_loop(0, _CH, edge_body, 0)
    _halve_ids2(db_, 0, dh, 2)
    pltpu.sync_copy(ubuf, acc_u.at[db_.at[0]], add=True)
    pltpu.sync_copy(dbuf, acc_d.at[dh.at[0]], add=True)

  prefetch(0, 0)

  def pair_body(p, carry):
    j0 = p * 2
    prefetch(j0 + 1, 1)
    consume(j0, 0)
    prefetch(j0 + 2, 0)
    consume(j0 + 1, 1)
    return carry

  lax.fori_loop(0, _NPAIR, pair_body, 0)
  drain(0)
  _copy_out(cid, sid, nblk, acc_u, acc_d, out_u, out_d, _RBD1)


@functools.partial(
    pl.kernel,
    name="sc_edge_l2",
    compiler_params=_SC_PARAMS,
    out_type=[
        jax.ShapeDtypeStruct((_NC, _N, 64), jnp.float32),
        jax.ShapeDtypeStruct((_NC, _ND2, 16), jnp.float32),
    ],
    mesh=_SC_MESH,
    scratch_types=[
        pltpu.VMEM((_CH, _D), jnp.float32),
        pltpu.VMEM((_CH, 192), jnp.float32),
        pltpu.VMEM((_CH, 192), jnp.float32),
        pltpu.VMEM((_CH, _D), jnp.float32),
        pltpu.VMEM((_CH, 192), jnp.float32),
        pltpu.VMEM((_CH, 192), jnp.float32),
        pltpu.VMEM((1, _CH), jnp.int32),
        pltpu.VMEM((1, _CH), jnp.int32),
        pltpu.VMEM((1, _CH), jnp.int32),
        pltpu.VMEM((1, _CH), jnp.int32),
        pltpu.VMEM((1, _CH), jnp.int32),
        pltpu.VMEM((_CH, 64), jnp.float32),
        pltpu.VMEM((_CH, 16), jnp.float32),
        pltpu.VMEM_SHARED((_N, 64), jnp.float32),
        pltpu.VMEM_SHARED((_ND2, 16), jnp.float32),
    ] + [pltpu.SemaphoreType.DMA] * 6,
)
def _sc_edge_l2(q_hbm, kv_hbm, ee_hbm, src_hbm, dst_hbm,
                out_u, out_d,
                qb0, kvb0, eb0, qb1, kvb1, eb1,
                sb0, db0, sb1, db1, dh, ubuf, dbuf, acc_u, acc_d,
                mq0, mk0, me0, mq1, mk1, me1):
  cid = lax.axis_index("c")
  sid = lax.axis_index("s")
  nblk = _zero_and_plan(sid, ubuf, dbuf, acc_u, acc_d, _RBD2)
  lane = lax.iota(jnp.int32, 16)
  lanem = lax.bitwise_and(lane, 1)
  laneq = lax.shift_right_logical(lane, 1)
  perms = _xor_perms(lane)
  zv = jnp.zeros((16,), jnp.float32)

  first = pl.multiple_of(sid * _CHT, 2)

  bufs = ((qb0, kvb0, eb0, sb0, db0, mq0, mk0, me0),
          (qb1, kvb1, eb1, sb1, db1, mq1, mk1, me1))

  def prefetch(j, b):
    qb, kvb, eb, sb, db_, mq, mk, me = bufs[b]
    jc = jnp.minimum(j, _CHT - 1)
    base = pl.multiple_of((first + jc) * _CH, 16)
    pltpu.sync_copy(src_hbm.at[pl.ds(first + jc, 1)], sb)
    pltpu.sync_copy(dst_hbm.at[pl.ds(first + jc, 1)], db_)
    pltpu.async_copy(q_hbm.at[db_.at[0]], qb, mq)
    pltpu.async_copy(kv_hbm.at[cid].at[sb.at[0]], kvb, mk)
    pltpu.async_copy(ee_hbm.at[cid].at[pl.ds(base, _CH)], eb, me)

  def drain(b):
    qb, kvb, eb, sb, db_, mq, mk, me = bufs[b]
    dummy_q = q_hbm.at[pl.ds(0, _CH)]
    dummy_kv = kv_hbm.at[cid].at[pl.ds(0, _CH)]
    pltpu.make_async_copy(dummy_q, qb, mq).wait()
    pltpu.make_async_copy(dummy_kv, kvb, mk).wait()
    pltpu.make_async_copy(dummy_kv, eb, me).wait()

  def consume(j, b):
    qb, kvb, eb, sb, db_, mq, mk, me = bufs[b]
    drain(b)

    def edge_body(i, ecarry):
      acc_t = zv
      for h in range(_D // 16):
        sl = pl.ds(16 * h, 16)
        kj = kvb[i, sl] + eb[i, sl]
        acc_t = acc_t + qb[i, sl] * kj
      aev = jnp.exp(_splat_sum(acc_t, perms))
      for h in range(64 // 16):
        slv = pl.ds(_D + 16 * h, 16)
        ubuf[i, pl.ds(16 * h, 16)] = (kvb[i, slv] + eb[i, slv]) * aev
      pv = _dst_slot_vec2(db_, 0, i, 7)
      dbuf[i, :] = jnp.where((lanem == 0) & (laneq == pv), aev, 0.0)
      return ecarry

    lax.fori_loop(0, _CH, edge_body, 0)
    _halve_ids2(db_, 0, dh, 3)
    pltpu.sync_copy(ubuf, acc_u.at[db_.at[0]], add=True)
    pltpu.sync_copy(dbuf, acc_d.at[dh.at[0]], add=True)

  prefetch(0, 0)

  def pair_body(p, carry):
    j0 = p * 2
    prefetch(j0 + 1, 1)
    consume(j0, 0)
    prefetch(j0 + 2, 0)
    consume(j0 + 1, 1)
    return carry

  lax.fori_loop(0, _NPAIR, pair_body, 0)
  drain(0)
  _copy_out(cid, sid, nblk, acc_u, acc_d, out_u, out_d, _RBD2)


def kernel(x, edge_index, edge_feats,
           Wq1, bq1, Wk1, bk1, Wv1, bv1, We1, be1, Ws1, bs1,
           Wq2, bq2, Wk2, bk2, Wv2, bv2, We2, be2, Ws2, bs2):
  scale1 = 1.0 / np.sqrt(np.float32(_HID))
  scale2 = 1.0 / np.sqrt(np.float32(_D))

  wcat1 = jnp.concatenate([Wq1 * scale1, Wk1, Wv1, Ws1], axis=1)
  bcat1 = jnp.concatenate([bq1 * scale1, bk1, bv1, bs1], axis=0)
  wcat2 = jnp.concatenate([Wq2 * scale2, Wk2, Wv2, Ws2], axis=1)
  bcat2 = jnp.concatenate([bq2 * scale2, bk2, bv2, bs2], axis=0)

  src2d = edge_index[0].reshape(_NCHUNK, _CH)
  dst2d = edge_index[1].reshape(_NCHUNK, _CH)

  heads_bm = np.zeros((16, _D), np.float32)
  for h in range(_HEADS):
    heads_bm[h, 16 * h:16 * (h + 1)] = 1.0
  heads_bm = jnp.asarray(heads_bm)
  ones_bm = np.zeros((16, _D), np.float32)
  ones_bm[0, :] = 1.0
  ones_bm = jnp.asarray(ones_bm)

  def split_cols(a):
    return jnp.stack([a[:, :64], a[:, 64:]])

  p1 = _matmul(x, wcat1, bcat1, 400)
  wst1 = jnp.stack([We1[:, :64], We1[:, 64:]])
  bst1 = jnp.stack([be1[:64].reshape(1, 64), be1[64:].reshape(1, 64)])
  ee1 = _ee_stacked_matmul(edge_feats, wst1, bst1, 2000)
  qs1 = split_cols(p1[:, 0:128])
  k1, v1 = p1[:, 128:256], p1[:, 256:384]
  kv1 = jnp.stack([jnp.concatenate([k1[:, :64], v1[:, :64]], axis=1),
                   jnp.concatenate([k1[:, 64:], v1[:, 64:]], axis=1)])
  s1 = p1[:, 384:512]

  u1, d1 = _sc_edge_l1(qs1, kv1, ee1, src2d, dst2d)
  uu1 = jnp.concatenate([u1[0], u1[1]], axis=1)
  d1r = d1.reshape(_NC, _N, 4)
  dd1 = jnp.concatenate(
      [d1r[0], d1r[1], jnp.zeros((_N, 8), jnp.float32)], axis=1)

  p2 = _epilogue_proj(uu1, dd1, s1, heads_bm, wcat2, bcat2, 400)
  wst2 = jnp.stack([jnp.concatenate([We2, We2[:, :64]], axis=1),
                    jnp.concatenate([We2, We2[:, 64:]], axis=1)])
  bst2 = jnp.stack([jnp.concatenate([be2, be2[:64]]).reshape(1, 192),
                    jnp.concatenate([be2, be2[64:]]).reshape(1, 192)])
  ee2 = _ee_stacked_matmul(edge_feats, wst2, bst2, 2000)
  q2 = p2[:, 0:128]
  k2, v2 = p2[:, 128:256], p2[:, 256:384]
  kv2 = jnp.stack([jnp.concatenate([k2, v2[:, :64]], axis=1),
                   jnp.concatenate([k2, v2[:, 64:]], axis=1)])
  s2 = p2[:, 384:512]

  u2, d2 = _sc_edge_l2(q2, kv2, ee2, src2d, dst2d)
  uu2 = jnp.concatenate([u2[0], u2[1]], axis=1)
  dd2 = jnp.concatenate(
      [d2.reshape(_NC, _N, 2)[0], jnp.zeros((_N, 14), jnp.float32)], axis=1)

  return _epilogue_final(uu2, dd2, s2, ones_bm, 400)


# speedup vs baseline: 1.3105x; 1.3105x over previous
import functools

import jax
import jax.numpy as jnp
import numpy as np
from jax import lax
from jax.experimental import pallas as pl
from jax.experimental.pallas import tpu as pltpu
from jax.experimental.pallas import tpu_sc as plsc

_N = 10000
_E = 320000
_D = 128
_HEADS = 8
_HID = 16

_NC = 2
_NS = 16
_CH = 128
_NCHUNK = _E // _CH
_RB = 80
_NRB = _N // _RB


def _mm_body(x_ref, w_ref, b_ref, o_ref):
  o_ref[...] = (
      jnp.dot(x_ref[...], w_ref[...], preferred_element_type=jnp.float32)
      + b_ref[...]
  )


def _matmul(x, w, b, block_rows):
  m, kdim = x.shape
  n = w.shape[1]
  return pl.pallas_call(
      _mm_body,
      grid=(m // block_rows,),
      in_specs=[
          pl.BlockSpec((block_rows, kdim), lambda i: (i, 0)),
          pl.BlockSpec((kdim, n), lambda i: (0, 0)),
          pl.BlockSpec((1, n), lambda i: (0, 0)),
      ],
      out_specs=pl.BlockSpec((block_rows, n), lambda i: (i, 0)),
      out_shape=jax.ShapeDtypeStruct((m, n), jnp.float32),
  )(x, w, b.reshape(1, n))


def _mm3_body(x_ref, w_ref, b_ref, o_ref):
  o_ref[0] = (
      jnp.dot(x_ref[...], w_ref[0], preferred_element_type=jnp.float32)
      + b_ref[0]
  )


def _ee_stacked_matmul(e, wst, bst, block_rows):
  m = e.shape[0]
  w_ = wst.shape[2]

  return pl.pallas_call(
      _mm3_body,
      grid=(_NC, m // block_rows),
      in_specs=[
          pl.BlockSpec((block_rows, 16), lambda g, i: (i, 0)),
          pl.BlockSpec((1, 16, w_), lambda g, i: (g, 0, 0)),
          pl.BlockSpec((1, 1, w_), lambda g, i: (g, 0, 0)),
      ],
      out_specs=pl.BlockSpec((1, block_rows, w_), lambda g, i: (g, i, 0)),
      out_shape=jax.ShapeDtypeStruct((_NC, m, w_), jnp.float32),
  )(e, wst, bst)


def _ep_body(u_ref, d_ref, s_ref, bm_ref, w_ref, b_ref, o_ref):
  dinv = 1.0 / jnp.maximum(d_ref[...], 1e-30)
  dbc = jnp.dot(dinv, bm_ref[...], preferred_element_type=jnp.float32)
  h = jnp.maximum(u_ref[...] * dbc + s_ref[...], 0.0)
  o_ref[...] = (
      jnp.dot(h, w_ref[...], preferred_element_type=jnp.float32) + b_ref[...]
  )


def _epilogue_proj(u, d, s, bmat, w, b, block_rows):
  m = u.shape[0]
  n = w.shape[1]
  return pl.pallas_call(
      _ep_body,
      grid=(m // block_rows,),
      in_specs=[
          pl.BlockSpec((block_rows, _D), lambda i: (i, 0)),
          pl.BlockSpec((block_rows, 16), lambda i: (i, 0)),
          pl.BlockSpec((block_rows, _D), lambda i: (i, 0)),
          pl.BlockSpec((16, _D), lambda i: (0, 0)),
          pl.BlockSpec((_D, n), lambda i: (0, 0)),
          pl.BlockSpec((1, n), lambda i: (0, 0)),
      ],
      out_specs=pl.BlockSpec((block_rows, n), lambda i: (i, 0)),
      out_shape=jax.ShapeDtypeStruct((m, n), jnp.float32),
  )(u, d, s, bmat, w, b.reshape(1, n))


def _ep_final_body(u_ref, d_ref, s_ref, bm_ref, o_ref):
  dinv = 1.0 / jnp.maximum(d_ref[...], 1e-30)
  dbc = jnp.dot(dinv, bm_ref[...], preferred_element_type=jnp.float32)
  o_ref[...] = jnp.maximum(u_ref[...] * dbc + s_ref[...], 0.0)


def _epilogue_final(u, d, s, bmat, block_rows):
  m = u.shape[0]
  return pl.pallas_call(
      _ep_final_body,
      grid=(m // block_rows,),
      in_specs=[
          pl.BlockSpec((block_rows, _D), lambda i: (i, 0)),
          pl.BlockSpec((block_rows, 16), lambda i: (i, 0)),
          pl.BlockSpec((block_rows, _D), lambda i: (i, 0)),
          pl.BlockSpec((16, _D), lambda i: (0, 0)),
      ],
      out_specs=pl.BlockSpec((block_rows, _D), lambda i: (i, 0)),
      out_shape=jax.ShapeDtypeStruct((m, _D), jnp.float32),
  )(u, d, s, bmat)


_SC_PARAMS = pltpu.CompilerParams(
    needs_layout_passes=False, use_tc_tiling_on_sc=False)
_SC_MESH = plsc.VectorSubcoreMesh(
    core_axis_name="c", subcore_axis_name="s",
    num_cores=_NC, num_subcores=_NS)
_ACC_SCRATCH = [
    pltpu.VMEM_SHARED((_N, 64), jnp.float32),
    pltpu.VMEM_SHARED((_N, 16), jnp.float32),
]
_SEMS = [pltpu.SemaphoreType.DMA] * 3


def _zero_and_plan(sid, ubuf, dbuf, acc_u, acc_d):
  zv = jnp.zeros((16,), jnp.float32)

  def zero_row(i, carry):
    for blk in range(64 // 16):
      ubuf[i, pl.ds(16 * blk, 16)] = zv
    dbuf[i, :] = zv
    return carry

  lax.fori_loop(0, _CH, zero_row, 0)
  nblk = jnp.where(sid < _NRB % _NS, _NRB // _NS + 1, _NRB // _NS)

  def zero_blk(j, carry):
    row = pl.multiple_of((sid + j * _NS) * _RB, _RB)
    pltpu.sync_copy(ubuf.at[pl.ds(0, _RB)], acc_u.at[pl.ds(row, _RB)])
    pltpu.sync_copy(dbuf.at[pl.ds(0, _RB)], acc_d.at[pl.ds(row, _RB)])
    return carry

  lax.fori_loop(0, nblk, zero_blk, 0)
  plsc.subcore_barrier()
  return nblk


def _copy_out(cid, sid, nblk, acc_u, acc_d, out_u, out_d):
  plsc.subcore_barrier()

  def out_blk(j, carry):
    row = pl.multiple_of((sid + j * _NS) * _RB, _RB)
    sl = pl.ds(row, _RB)
    pltpu.sync_copy(acc_u.at[sl], out_u.at[cid, sl])
    pltpu.sync_copy(acc_d.at[sl], out_d.at[cid, sl])
    return carry

  lax.fori_loop(0, nblk, out_blk, 0)


@functools.partial(
    pl.kernel,
    name="sc_edge_l1",
    compiler_params=_SC_PARAMS,
    out_type=[
        jax.ShapeDtypeStruct((_NC, _N, 64), jnp.float32),
        jax.ShapeDtypeStruct((_NC, _N, 16), jnp.float32),
    ],
    mesh=_SC_MESH,
    scratch_types=[
        pltpu.VMEM((_CH, 64), jnp.float32),
        pltpu.VMEM((_CH, 64), jnp.float32),
        pltpu.VMEM((_CH, 64), jnp.float32),
        pltpu.VMEM((_CH, 64), jnp.float32),
        pltpu.VMEM((_CH, 64), jnp.float32),
        pltpu.VMEM((_CH, 16), jnp.float32),
        pltpu.VMEM((1, _CH), jnp.int32),
        pltpu.VMEM((1, _CH), jnp.int32),
    ] + _ACC_SCRATCH + _SEMS,
)
def _sc_edge_l1(q_hbm, k_hbm, v_hbm, ee_hbm, src_hbm, dst_hbm,
                out_u, out_d,
                qbuf, kbuf, vbuf, eebuf, ubuf, dbuf, srcbuf, dstbuf,
                acc_u, acc_d, sem_q, sem_k, sem_v):
  cid = lax.axis_index("c")
  sid = lax.axis_index("s")
  nblk = _zero_and_plan(sid, ubuf, dbuf, acc_u, acc_d)
  lane = lax.iota(jnp.int32, 16)
  zv = jnp.zeros((16,), jnp.float32)

  nch = jnp.where(sid < _NCHUNK % _NS, _NCHUNK // _NS + 1, _NCHUNK // _NS)

  def chunk_body(j, carry):
    chunk = sid + j * _NS
    base = pl.multiple_of(chunk * _CH, _CH)
    pltpu.sync_copy(src_hbm.at[pl.ds(chunk, 1)], srcbuf)
    pltpu.sync_copy(dst_hbm.at[pl.ds(chunk, 1)], dstbuf)
    cq = pltpu.async_copy(q_hbm.at[cid].at[dstbuf.at[0]], qbuf, sem_q)
    ck = pltpu.async_copy(k_hbm.at[cid].at[srcbuf.at[0]], kbuf, sem_k)
    cv = pltpu.async_copy(v_hbm.at[cid].at[srcbuf.at[0]], vbuf, sem_v)
    pltpu.sync_copy(ee_hbm.at[cid].at[pl.ds(base, _CH)], eebuf)
    cq.wait()
    ck.wait()
    cv.wait()

    def edge_body(i, ecarry):
      dacc = zv
      for h in range(_HEADS // _NC):
        sl = pl.ds(16 * h, 16)
        kj = kbuf[i, sl] + eebuf[i, sl]
        s = jnp.sum(qbuf[i, sl] * kj)
        aev = jnp.exp(jnp.full((16,), s, jnp.float32))
        ubuf[i, sl] = (vbuf[i, sl] + eebuf[i, sl]) * aev
        dacc = dacc + jnp.where(lane == h, aev, 0.0)
      dbuf[i, :] = dacc
      return ecarry

    lax.fori_loop(0, _CH, edge_body, 0)
    pltpu.sync_copy(ubuf, acc_u.at[dstbuf.at[0]], add=True)
    pltpu.sync_copy(dbuf, acc_d.at[dstbuf.at[0]], add=True)
    return carry

  lax.fori_loop(0, nch, chunk_body, 0)
  _copy_out(cid, sid, nblk, acc_u, acc_d, out_u, out_d)


@functools.partial(
    pl.kernel,
    name="sc_edge_l2",
    compiler_params=_SC_PARAMS,
    out_type=[
        jax.ShapeDtypeStruct((_NC, _N, 64), jnp.float32),
        jax.ShapeDtypeStruct((_NC, _N, 16), jnp.float32),
    ],
    mesh=_SC_MESH,
    scratch_types=[
        pltpu.VMEM((_CH, _D), jnp.float32),
        pltpu.VMEM((_CH, _D), jnp.float32),
        pltpu.VMEM((_CH, 64), jnp.float32),
        pltpu.VMEM((_CH, _D), jnp.float32),
        pltpu.VMEM((_CH, 64), jnp.float32),
        pltpu.VMEM((_CH, 64), jnp.float32),
        pltpu.VMEM((_CH, 16), jnp.float32),
        pltpu.VMEM((1, _CH), jnp.int32),
        pltpu.VMEM((1, _CH), jnp.int32),
    ] + _ACC_SCRATCH + _SEMS,
)
def _sc_edge_l2(q_hbm, k_hbm, v_hbm, eef_hbm, ees_hbm, src_hbm, dst_hbm,
                out_u, out_d,
                qbuf, kbuf, vbuf, eebuf, eehbuf, ubuf, dbuf, srcbuf, dstbuf,
                acc_u, acc_d, sem_q, sem_k, sem_v):
  cid = lax.axis_index("c")
  sid = lax.axis_index("s")
  nblk = _zero_and_plan(sid, ubuf, dbuf, acc_u, acc_d)
  lane = lax.iota(jnp.int32, 16)
  zv = jnp.zeros((16,), jnp.float32)

  nch = jnp.where(sid < _NCHUNK % _NS, _NCHUNK // _NS + 1, _NCHUNK // _NS)

  def chunk_body(j, carry):
    chunk = sid + j * _NS
    base = pl.multiple_of(chunk * _CH, _CH)
    pltpu.sync_copy(src_hbm.at[pl.ds(chunk, 1)], srcbuf)
    pltpu.sync_copy(dst_hbm.at[pl.ds(chunk, 1)], dstbuf)
    cq = pltpu.async_copy(q_hbm.at[dstbuf.at[0]], qbuf, sem_q)
    ck = pltpu.async_copy(k_hbm.at[srcbuf.at[0]], kbuf, sem_k)
    cv = pltpu.async_copy(v_hbm.at[cid].at[srcbuf.at[0]], vbuf, sem_v)
    pltpu.sync_copy(eef_hbm.at[pl.ds(base, _CH)], eebuf)
    pltpu.sync_copy(ees_hbm.at[cid].at[pl.ds(base, _CH)], eehbuf)
    cq.wait()
    ck.wait()
    cv.wait()

    def edge_body(i, ecarry):
      acc_t = zv
      for h in range(_D // 16):
        sl = pl.ds(16 * h, 16)
        kj = kbuf[i, sl] + eebuf[i, sl]
        acc_t = acc_t + qbuf[i, sl] * kj
      s = jnp.sum(acc_t)
      aev = jnp.exp(jnp.full((16,), s, jnp.float32))
      for h in range(64 // 16):
        sl = pl.ds(16 * h, 16)
        ubuf[i, sl] = (vbuf[i, sl] + eehbuf[i, sl]) * aev
      dbuf[i, :] = jnp.where(lane == 0, aev, 0.0)
      return ecarry

    lax.fori_loop(0, _CH, edge_body, 0)
    pltpu.sync_copy(ubuf, acc_u.at[dstbuf.at[0]], add=True)
    pltpu.sync_copy(dbuf, acc_d.at[dstbuf.at[0]], add=True)
    return carry

  lax.fori_loop(0, nch, chunk_body, 0)
  _copy_out(cid, sid, nblk, acc_u, acc_d, out_u, out_d)


def kernel(x, edge_index, edge_feats,
           Wq1, bq1, Wk1, bk1, Wv1, bv1, We1, be1, Ws1, bs1,
           Wq2, bq2, Wk2, bk2, Wv2, bv2, We2, be2, Ws2, bs2):
  scale1 = 1.0 / np.sqrt(np.float32(_HID))
  scale2 = 1.0 / np.sqrt(np.float32(_D))

  wcat1 = jnp.concatenate([Wq1 * scale1, Wk1, Wv1, Ws1], axis=1)
  bcat1 = jnp.concatenate([bq1 * scale1, bk1, bv1, bs1], axis=0)
  wcat2 = jnp.concatenate([Wq2 * scale2, Wk2, Wv2, Ws2], axis=1)
  bcat2 = jnp.concatenate([bq2 * scale2, bk2, bv2, bs2], axis=0)

  src2d = edge_index[0].reshape(_NCHUNK, _CH)
  dst2d = edge_index[1].reshape(_NCHUNK, _CH)

  heads_bm = np.zeros((16, _D), np.float32)
  for h in range(_HEADS):
    heads_bm[h, 16 * h:16 * (h + 1)] = 1.0
  heads_bm = jnp.asarray(heads_bm)
  ones_bm = np.zeros((16, _D), np.float32)
  ones_bm[0, :] = 1.0
  ones_bm = jnp.asarray(ones_bm)

  def split_cols(a):
    return jnp.stack([a[:, :64], a[:, 64:]])

  def split_stack(w, b):
    wst = jnp.stack([w[:, :64], w[:, 64:]])
    bst = jnp.stack([b[:64].reshape(1, 64), b[64:].reshape(1, 64)])
    return wst, bst

  p1 = _matmul(x, wcat1, bcat1, 400)
  wst1, bst1 = split_stack(We1, be1)
  ee1 = _ee_stacked_matmul(edge_feats, wst1, bst1, 2000)
  qs1 = split_cols(p1[:, 0:128])
  ks1 = split_cols(p1[:, 128:256])
  vs1 = split_cols(p1[:, 256:384])
  s1 = p1[:, 384:512]

  u1, d1 = _sc_edge_l1(qs1, ks1, vs1, ee1, src2d, dst2d)
  uu1 = jnp.concatenate([u1[0], u1[1]], axis=1)
  dd1 = jnp.concatenate(
      [d1[0, :, :4], d1[1, :, :4], jnp.zeros((_N, 8), jnp.float32)], axis=1)

  p2 = _epilogue_proj(uu1, dd1, s1, heads_bm, wcat2, bcat2, 400)
  ee2f = _matmul(edge_feats, We2, be2, 2000)
  wst2, bst2 = split_stack(We2, be2)
  ee2s = _ee_stacked_matmul(edge_feats, wst2, bst2, 2000)
  q2 = p2[:, 0:128]
  k2 = p2[:, 128:256]
  vs2 = split_cols(p2[:, 256:384])
  s2 = p2[:, 384:512]

  u2, d2 = _sc_edge_l2(q2, k2, vs2, ee2f, ee2s, src2d, dst2d)
  uu2 = jnp.concatenate([u2[0], u2[1]], axis=1)

  return _epilogue_final(uu2, d2[0], s2, ones_bm, 400)


# speedup vs baseline: 1.3518x; 1.0315x over previous
import functools

import jax
import jax.numpy as jnp
import numpy as np
from jax import lax
from jax.experimental import pallas as pl
from jax.experimental.pallas import tpu as pltpu
from jax.experimental.pallas import tpu_sc as plsc

_N = 10000
_E = 320000
_D = 128
_HEADS = 8
_HID = 16

_NC = 2
_NS = 16
_CH = 128
_NCHUNK = _E // _CH
_RB = 80
_NRB = _N // _RB


def _mm_body(x_ref, w_ref, b_ref, o_ref):
  o_ref[...] = (
      jnp.dot(x_ref[...], w_ref[...], preferred_element_type=jnp.float32)
      + b_ref[...]
  )


def _matmul(x, w, b, block_rows):
  m, kdim = x.shape
  n = w.shape[1]
  return pl.pallas_call(
      _mm_body,
      grid=(m // block_rows,),
      in_specs=[
          pl.BlockSpec((block_rows, kdim), lambda i: (i, 0)),
          pl.BlockSpec((kdim, n), lambda i: (0, 0)),
          pl.BlockSpec((1, n), lambda i: (0, 0)),
      ],
      out_specs=pl.BlockSpec((block_rows, n), lambda i: (i, 0)),
      out_shape=jax.ShapeDtypeStruct((m, n), jnp.float32),
  )(x, w, b.reshape(1, n))


def _mm3_body(x_ref, w_ref, b_ref, o_ref):
  o_ref[0] = (
      jnp.dot(x_ref[...], w_ref[0], preferred_element_type=jnp.float32)
      + b_ref[0]
  )


def _ee_stacked_matmul(e, wst, bst, block_rows):
  m = e.shape[0]
  w_ = wst.shape[2]

  return pl.pallas_call(
      _mm3_body,
      grid=(_NC, m // block_rows),
      in_specs=[
          pl.BlockSpec((block_rows, 16), lambda g, i: (i, 0)),
          pl.BlockSpec((1, 16, w_), lambda g, i: (g, 0, 0)),
          pl.BlockSpec((1, 1, w_), lambda g, i: (g, 0, 0)),
      ],
      out_specs=pl.BlockSpec((1, block_rows, w_), lambda g, i: (g, i, 0)),
      out_shape=jax.ShapeDtypeStruct((_NC, m, w_), jnp.float32),
  )(e, wst, bst)


def _ep_body(u_ref, d_ref, s_ref, bm_ref, w_ref, b_ref, o_ref):
  dinv = 1.0 / jnp.maximum(d_ref[...], 1e-30)
  dbc = jnp.dot(dinv, bm_ref[...], preferred_element_type=jnp.float32)
  h = jnp.maximum(u_ref[...] * dbc + s_ref[...], 0.0)
  o_ref[...] = (
      jnp.dot(h, w_ref[...], preferred_element_type=jnp.float32) + b_ref[...]
  )


def _epilogue_proj(u, d, s, bmat, w, b, block_rows):
  m = u.shape[0]
  n = w.shape[1]
  return pl.pallas_call(
      _ep_body,
      grid=(m // block_rows,),
      in_specs=[
          pl.BlockSpec((block_rows, _D), lambda i: (i, 0)),
          pl.BlockSpec((block_rows, 16), lambda i: (i, 0)),
          pl.BlockSpec((block_rows, _D), lambda i: (i, 0)),
          pl.BlockSpec((16, _D), lambda i: (0, 0)),
          pl.BlockSpec((_D, n), lambda i: (0, 0)),
          pl.BlockSpec((1, n), lambda i: (0, 0)),
      ],
      out_specs=pl.BlockSpec((block_rows, n), lambda i: (i, 0)),
      out_shape=jax.ShapeDtypeStruct((m, n), jnp.float32),
  )(u, d, s, bmat, w, b.reshape(1, n))


def _ep_final_body(u_ref, d_ref, s_ref, bm_ref, o_ref):
  dinv = 1.0 / jnp.maximum(d_ref[...], 1e-30)
  dbc = jnp.dot(dinv, bm_ref[...], preferred_element_type=jnp.float32)
  o_ref[...] = jnp.maximum(u_ref[...] * dbc + s_ref[...], 0.0)


def _epilogue_final(u, d, s, bmat, block_rows):
  m = u.shape[0]
  return pl.pallas_call(
      _ep_final_body,
      grid=(m // block_rows,),
      in_specs=[
          pl.BlockSpec((block_rows, _D), lambda i: (i, 0)),
          pl.BlockSpec((block_rows, 16), lambda i: (i, 0)),
          pl.BlockSpec((block_rows, _D), lambda i: (i, 0)),
          pl.BlockSpec((16, _D), lambda i: (0, 0)),
      ],
      out_specs=pl.BlockSpec((block_rows, _D), lambda i: (i, 0)),
      out_shape=jax.ShapeDtypeStruct((m, _D), jnp.float32),
  )(u, d, s, bmat)


_SC_PARAMS = pltpu.CompilerParams(
    needs_layout_passes=False, use_tc_tiling_on_sc=False)
_SC_MESH = plsc.VectorSubcoreMesh(
    core_axis_name="c", subcore_axis_name="s",
    num_cores=_NC, num_subcores=_NS)
_ACC_SCRATCH = [
    pltpu.VMEM_SHARED((_N, 64), jnp.float32),
    pltpu.VMEM_SHARED((_N, 16), jnp.float32),
]
_SEMS = [pltpu.SemaphoreType.DMA] * 5


def _zero_and_plan(sid, ubuf, dbuf, acc_u, acc_d):
  zv = jnp.zeros((16,), jnp.float32)

  def zero_row(i, carry):
    for blk in range(64 // 16):
      ubuf[i, pl.ds(16 * blk, 16)] = zv
    dbuf[i, :] = zv
    return carry

  lax.fori_loop(0, _CH, zero_row, 0)
  nblk = jnp.where(sid < _NRB % _NS, _NRB // _NS + 1, _NRB // _NS)

  def zero_blk(j, carry):
    row = pl.multiple_of((sid + j * _NS) * _RB, _RB)
    pltpu.sync_copy(ubuf.at[pl.ds(0, _RB)], acc_u.at[pl.ds(row, _RB)])
    pltpu.sync_copy(dbuf.at[pl.ds(0, _RB)], acc_d.at[pl.ds(row, _RB)])
    return carry

  lax.fori_loop(0, nblk, zero_blk, 0)
  plsc.subcore_barrier()
  return nblk


def _copy_out(cid, sid, nblk, acc_u, acc_d, out_u, out_d):
  plsc.subcore_barrier()

  def out_blk(j, carry):
    row = pl.multiple_of((sid + j * _NS) * _RB, _RB)
    sl = pl.ds(row, _RB)
    pltpu.sync_copy(acc_u.at[sl], out_u.at[cid, sl])
    pltpu.sync_copy(acc_d.at[sl], out_d.at[cid, sl])
    return carry

  lax.fori_loop(0, nblk, out_blk, 0)


@functools.partial(
    pl.kernel,
    name="sc_edge_l1",
    compiler_params=_SC_PARAMS,
    out_type=[
        jax.ShapeDtypeStruct((_NC, _N, 64), jnp.float32),
        jax.ShapeDtypeStruct((_NC, _N, 16), jnp.float32),
    ],
    mesh=_SC_MESH,
    scratch_types=[
        pltpu.VMEM((_CH, 64), jnp.float32),
        pltpu.VMEM((_CH, 64), jnp.float32),
        pltpu.VMEM((_CH, 64), jnp.float32),
        pltpu.VMEM((_CH, 64), jnp.float32),
        pltpu.VMEM((_CH, 64), jnp.float32),
        pltpu.VMEM((_CH, 16), jnp.float32),
        pltpu.VMEM((1, _CH), jnp.int32),
        pltpu.VMEM((1, _CH), jnp.int32),
    ] + _ACC_SCRATCH + _SEMS,
)
def _sc_edge_l1(q_hbm, k_hbm, v_hbm, ee_hbm, src_hbm, dst_hbm,
                out_u, out_d,
                qbuf, kbuf, vbuf, eebuf, ubuf, dbuf, srcbuf, dstbuf,
                acc_u, acc_d, sem_q, sem_k, sem_v, sem_e, sem_e2):
  cid = lax.axis_index("c")
  sid = lax.axis_index("s")
  nblk = _zero_and_plan(sid, ubuf, dbuf, acc_u, acc_d)
  lane = lax.iota(jnp.int32, 16)
  zv = jnp.zeros((16,), jnp.float32)

  nch = jnp.where(sid < _NCHUNK % _NS, _NCHUNK // _NS + 1, _NCHUNK // _NS)

  def chunk_body(j, carry):
    chunk = sid + j * _NS
    base = pl.multiple_of(chunk * _CH, _CH)
    pltpu.sync_copy(src_hbm.at[pl.ds(chunk, 1)], srcbuf)
    pltpu.sync_copy(dst_hbm.at[pl.ds(chunk, 1)], dstbuf)
    cq = pltpu.async_copy(q_hbm.at[cid].at[dstbuf.at[0]], qbuf, sem_q)
    ck = pltpu.async_copy(k_hbm.at[cid].at[srcbuf.at[0]], kbuf, sem_k)
    cv = pltpu.async_copy(v_hbm.at[cid].at[srcbuf.at[0]], vbuf, sem_v)
    ce = pltpu.async_copy(ee_hbm.at[cid].at[pl.ds(base, _CH)], eebuf, sem_e)
    cq.wait()
    ck.wait()
    cv.wait()
    ce.wait()

    def edge_body(i, ecarry):
      dacc = zv
      for h in range(_HEADS // _NC):
        sl = pl.ds(16 * h, 16)
        kj = kbuf[i, sl] + eebuf[i, sl]
        s = jnp.sum(qbuf[i, sl] * kj)
        aev = jnp.exp(jnp.full((16,), s, jnp.float32))
        ubuf[i, sl] = (vbuf[i, sl] + eebuf[i, sl]) * aev
        dacc = dacc + jnp.where(lane == h, aev, 0.0)
      dbuf[i, :] = dacc
      return ecarry

    lax.fori_loop(0, _CH, edge_body, 0)
    pltpu.sync_copy(ubuf, acc_u.at[dstbuf.at[0]], add=True)
    pltpu.sync_copy(dbuf, acc_d.at[dstbuf.at[0]], add=True)
    return carry

  lax.fori_loop(0, nch, chunk_body, 0)
  _copy_out(cid, sid, nblk, acc_u, acc_d, out_u, out_d)


@functools.partial(
    pl.kernel,
    name="sc_edge_l2",
    compiler_params=_SC_PARAMS,
    out_type=[
        jax.ShapeDtypeStruct((_NC, _N, 64), jnp.float32),
        jax.ShapeDtypeStruct((_NC, _N, 16), jnp.float32),
    ],
    mesh=_SC_MESH,
    scratch_types=[
        pltpu.VMEM((_CH, _D), jnp.float32),
        pltpu.VMEM((_CH, _D), jnp.float32),
        pltpu.VMEM((_CH, 64), jnp.float32),
        pltpu.VMEM((_CH, _D), jnp.float32),
        pltpu.VMEM((_CH, 64), jnp.float32),
        pltpu.VMEM((_CH, 64), jnp.float32),
        pltpu.VMEM((_CH, 16), jnp.float32),
        pltpu.VMEM((1, _CH), jnp.int32),
        pltpu.VMEM((1, _CH), jnp.int32),
    ] + _ACC_SCRATCH + _SEMS,
)
def _sc_edge_l2(q_hbm, k_hbm, v_hbm, eef_hbm, ees_hbm, src_hbm, dst_hbm,
                out_u, out_d,
                qbuf, kbuf, vbuf, eebuf, eehbuf, ubuf, dbuf, srcbuf, dstbuf,
                acc_u, acc_d, sem_q, sem_k, sem_v, sem_e, sem_e2):
  cid = lax.axis_index("c")
  sid = lax.axis_index("s")
  nblk = _zero_and_plan(sid, ubuf, dbuf, acc_u, acc_d)
  lane = lax.iota(jnp.int32, 16)
  zv = jnp.zeros((16,), jnp.float32)

  nch = jnp.where(sid < _NCHUNK % _NS, _NCHUNK // _NS + 1, _NCHUNK // _NS)

  def chunk_body(j, carry):
    chunk = sid + j * _NS
    base = pl.multiple_of(chunk * _CH, _CH)
    pltpu.sync_copy(src_hbm.at[pl.ds(chunk, 1)], srcbuf)
    pltpu.sync_copy(dst_hbm.at[pl.ds(chunk, 1)], dstbuf)
    cq = pltpu.async_copy(q_hbm.at[dstbuf.at[0]], qbuf, sem_q)
    ck = pltpu.async_copy(k_hbm.at[srcbuf.at[0]], kbuf, sem_k)
    cv = pltpu.async_copy(v_hbm.at[cid].at[srcbuf.at[0]], vbuf, sem_v)
    ce = pltpu.async_copy(eef_hbm.at[pl.ds(base, _CH)], eebuf, sem_e)
    ce2 = pltpu.async_copy(ees_hbm.at[cid].at[pl.ds(base, _CH)], eehbuf, sem_e2)
    cq.wait()
    ck.wait()
    cv.wait()
    ce.wait()
    ce2.wait()

    def edge_body(i, ecarry):
      acc_t = zv
      for h in range(_D // 16):
        sl = pl.ds(16 * h, 16)
        kj = kbuf[i, sl] + eebuf[i, sl]
        acc_t = acc_t + qbuf[i, sl] * kj
      s = jnp.sum(acc_t)
      aev = jnp.exp(jnp.full((16,), s, jnp.float32))
      for h in range(64 // 16):
        sl = pl.ds(16 * h, 16)
        ubuf[i, sl] = (vbuf[i, sl] + eehbuf[i, sl]) * aev
      dbuf[i, :] = jnp.where(lane == 0, aev, 0.0)
      return ecarry

    lax.fori_loop(0, _CH, edge_body, 0)
    pltpu.sync_copy(ubuf, acc_u.at[dstbuf.at[0]], add=True)
    pltpu.sync_copy(dbuf, acc_d.at[dstbuf.at[0]], add=True)
    return carry

  lax.fori_loop(0, nch, chunk_body, 0)
  _copy_out(cid, sid, nblk, acc_u, acc_d, out_u, out_d)


def kernel(x, edge_index, edge_feats,
           Wq1, bq1, Wk1, bk1, Wv1, bv1, We1, be1, Ws1, bs1,
           Wq2, bq2, Wk2, bk2, Wv2, bv2, We2, be2, Ws2, bs2):
  scale1 = 1.0 / np.sqrt(np.float32(_HID))
  scale2 = 1.0 / np.sqrt(np.float32(_D))

  wcat1 = jnp.concatenate([Wq1 * scale1, Wk1, Wv1, Ws1], axis=1)
  bcat1 = jnp.concatenate([bq1 * scale1, bk1, bv1, bs1], axis=0)
  wcat2 = jnp.concatenate([Wq2 * scale2, Wk2, Wv2, Ws2], axis=1)
  bcat2 = jnp.concatenate([bq2 * scale2, bk2, bv2, bs2], axis=0)

  src2d = edge_index[0].reshape(_NCHUNK, _CH)
  dst2d = edge_index[1].reshape(_NCHUNK, _CH)

  heads_bm = np.zeros((16, _D), np.float32)
  for h in range(_HEADS):
    heads_bm[h, 16 * h:16 * (h + 1)] = 1.0
  heads_bm = jnp.asarray(heads_bm)
  ones_bm = np.zeros((16, _D), np.float32)
  ones_bm[0, :] = 1.0
  ones_bm = jnp.asarray(ones_bm)

  def split_cols(a):
    return jnp.stack([a[:, :64], a[:, 64:]])

  def split_stack(w, b):
    wst = jnp.stack([w[:, :64], w[:, 64:]])
    bst = jnp.stack([b[:64].reshape(1, 64), b[64:].reshape(1, 64)])
    return wst, bst

  p1 = _matmul(x, wcat1, bcat1, 400)
  wst1, bst1 = split_stack(We1, be1)
  ee1 = _ee_stacked_matmul(edge_feats, wst1, bst1, 2000)
  qs1 = split_cols(p1[:, 0:128])
  ks1 = split_cols(p1[:, 128:256])
  vs1 = split_cols(p1[:, 256:384])
  s1 = p1[:, 384:512]

  u1, d1 = _sc_edge_l1(qs1, ks1, vs1, ee1, src2d, dst2d)
  uu1 = jnp.concatenate([u1[0], u1[1]], axis=1)
  dd1 = jnp.concatenate(
      [d1[0, :, :4], d1[1, :, :4], jnp.zeros((_N, 8), jnp.float32)], axis=1)

  p2 = _epilogue_proj(uu1, dd1, s1, heads_bm, wcat2, bcat2, 400)
  ee2f = _matmul(edge_feats, We2, be2, 2000)
  wst2, bst2 = split_stack(We2, be2)
  ee2s = _ee_stacked_matmul(edge_feats, wst2, bst2, 2000)
  q2 = p2[:, 0:128]
  k2 = p2[:, 128:256]
  vs2 = split_cols(p2[:, 256:384])
  s2 = p2[:, 384:512]

  u2, d2 = _sc_edge_l2(q2, k2, vs2, ee2f, ee2s, src2d, dst2d)
  uu2 = jnp.concatenate([u2[0], u2[1]], axis=1)

  return _epilogue_final(uu2, d2[0], s2, ones_bm, 400)
